# Initial kernel scaffold; baseline (speedup 1.0000x reference)
import jax
import jax.numpy as jnp
from jax.experimental import pallas as pl


def kernel(table, token_ids):
    raise NotImplementedError("write your pallas kernel here")


# speedup vs baseline: 2.3295x; 2.3295x over previous
import functools

import jax
import jax.numpy as jnp
from jax import lax
from jax.experimental import pallas as pl
from jax.experimental.pallas import tpu as pltpu
from jax.experimental.pallas import tpu_sc as plsc

VOCAB = 50
DIM = 16
BATCH = 4096
FIELDS = 26

_NC = 2
_NS = 16
_NW = _NC * _NS

_TOKENS = BATCH * FIELDS
_TPW = _TOKENS // _NW
_CHUNK = 128
_CPW = _TPW // _CHUNK
_NCHUNKS = _TOKENS // _CHUNK


@functools.partial(
    pl.kernel,
    out_type=jax.ShapeDtypeStruct((_NCHUNKS, _CHUNK, DIM), jnp.float32),
    mesh=plsc.VectorSubcoreMesh(core_axis_name="c", subcore_axis_name="s"),
    scratch_types=[
        pltpu.VMEM((_TPW,), jnp.int32),
        pltpu.VMEM((_CPW, _CHUNK, DIM), jnp.float32),
        pltpu.SemaphoreType.DMA,
    ],
    compiler_params=pltpu.CompilerParams(use_tc_tiling_on_sc=False),
)
def _gather_kernel(table_hbm, idx_hbm, out_hbm, idx_v, rows_v, sem):
    wid = lax.axis_index("s") * _NC + lax.axis_index("c")
    pltpu.sync_copy(idx_hbm.at[pl.ds(wid * _TPW, _TPW)], idx_v)
    copies = []
    for j in range(_CPW):
        copies.append(
            pltpu.async_copy(
                table_hbm.at[idx_v.at[pl.ds(j * _CHUNK, _CHUNK)]],
                rows_v.at[j], sem))
    for c in copies:
        c.wait()
    pltpu.sync_copy(rows_v, out_hbm.at[pl.ds(wid * _CPW, _CPW)])


def kernel(table, token_ids):
    idx = token_ids.reshape(_TOKENS)
    out = _gather_kernel(table, idx)
    return out.reshape(BATCH, FIELDS, DIM)


# speedup vs baseline: 2.7474x; 1.1794x over previous
import functools

import jax
import jax.numpy as jnp
from jax import lax
from jax.experimental import pallas as pl
from jax.experimental.pallas import tpu as pltpu
from jax.experimental.pallas import tpu_sc as plsc

VOCAB = 50
DIM = 16
BATCH = 4096
FIELDS = 26

_NC = 2
_NS = 16
_NW = _NC * _NS

_TOKENS = BATCH * FIELDS
_TPW = _TOKENS // _NW
_GROUPS = _TPW // DIM


@functools.partial(
    pl.kernel,
    out_type=jax.ShapeDtypeStruct((_TOKENS, DIM), jnp.float32),
    mesh=plsc.VectorSubcoreMesh(core_axis_name="c", subcore_axis_name="s"),
    scratch_types=[
        pltpu.VMEM((VOCAB, DIM), jnp.float32),
        pltpu.VMEM((_TPW,), jnp.int32),
        pltpu.VMEM((_TPW, DIM), jnp.float32),
    ],
    compiler_params=pltpu.CompilerParams(
        use_tc_tiling_on_sc=False, needs_layout_passes=False),
)
def _gather_kernel(table_hbm, idx_hbm, out_hbm, tab_v, idx_v, out_v):
    wid = lax.axis_index("s") * _NC + lax.axis_index("c")
    base = wid * _TPW
    pltpu.sync_copy(table_hbm, tab_v)
    pltpu.sync_copy(idx_hbm.at[pl.ds(base, _TPW)], idx_v)

    lanes = lax.iota(jnp.int32, DIM)
    cols = [jnp.full((DIM,), c, jnp.int32) for c in range(DIM)]

    def group(g, carry):
        v = idx_v[pl.ds(g * DIM, DIM)]
        row = g * DIM + lanes
        for c in range(DIM):
            w = plsc.load_gather(tab_v, [v, cols[c]])
            plsc.store_scatter(out_v, [row, cols[c]], w)
        return carry

    lax.fori_loop(0, _GROUPS, group, 0)
    pltpu.sync_copy(out_v, out_hbm.at[pl.ds(base, _TPW)])


def kernel(table, token_ids):
    idx = token_ids.reshape(_TOKENS)
    out = _gather_kernel(table, idx)
    return out.reshape(BATCH, FIELDS, DIM)


# speedup vs baseline: 3.9638x; 1.4428x over previous
import functools

import jax
import jax.numpy as jnp
from jax import lax
from jax.experimental import pallas as pl
from jax.experimental.pallas import tpu as pltpu
from jax.experimental.pallas import tpu_sc as plsc

VOCAB = 50
DIM = 16
BATCH = 4096
FIELDS = 26

_NC = 2
_NS = 16
_NW = _NC * _NS

_RPW = BATCH // _NW
_TPW = _RPW * FIELDS
_GROUPS = _TPW // DIM


@functools.partial(
    pl.kernel,
    out_type=jax.ShapeDtypeStruct((BATCH, FIELDS, DIM), jnp.float32),
    mesh=plsc.VectorSubcoreMesh(core_axis_name="c", subcore_axis_name="s"),
    scratch_types=[
        pltpu.VMEM((VOCAB, DIM), jnp.float32),
        pltpu.VMEM((_RPW, FIELDS), jnp.int32),
        pltpu.VMEM((_RPW, FIELDS, DIM), jnp.float32),
    ],
    compiler_params=pltpu.CompilerParams(
        use_tc_tiling_on_sc=False, needs_layout_passes=False),
)
def _gather_kernel(table_hbm, idx_hbm, out_hbm, tab_v, idx_v, out_v):
    wid = lax.axis_index("s") * _NC + lax.axis_index("c")
    base = wid * _RPW
    pltpu.sync_copy(table_hbm, tab_v)
    pltpu.sync_copy(idx_hbm.at[pl.ds(base, _RPW)], idx_v)

    lanes = lax.iota(jnp.int32, DIM)
    zero = jnp.zeros((DIM,), jnp.int32)
    cols = [jnp.full((DIM,), c, jnp.int32) for c in range(DIM)]

    def group(g, carry):
        t = g * DIM + lanes
        v = plsc.load_gather(idx_v, [zero, t])
        for c in range(DIM):
            w = plsc.load_gather(tab_v, [v, cols[c]])
            plsc.store_scatter(out_v, [zero, t, cols[c]], w)
        return carry

    lax.fori_loop(0, _GROUPS, group, 0)
    pltpu.sync_copy(out_v, out_hbm.at[pl.ds(base, _RPW)])


def kernel(table, token_ids):
    return _gather_kernel(table, token_ids)


# speedup vs baseline: 4.6399x; 1.1706x over previous
import functools

import jax
import jax.numpy as jnp
from jax import lax
from jax.experimental import pallas as pl
from jax.experimental.pallas import tpu as pltpu
from jax.experimental.pallas import tpu_sc as plsc

VOCAB = 50
DIM = 16
BATCH = 4096
FIELDS = 26

_NC = 2
_NS = 16
_NW = _NC * _NS

_RPW = BATCH // _NW
_TPW = _RPW * FIELDS
_GROUPS = _TPW // DIM


@functools.partial(
    pl.kernel,
    out_type=jax.ShapeDtypeStruct((BATCH, FIELDS, DIM), jnp.float32),
    mesh=plsc.VectorSubcoreMesh(core_axis_name="c", subcore_axis_name="s"),
    scratch_types=[
        pltpu.VMEM((VOCAB, DIM), jnp.float32),
        pltpu.VMEM((_RPW, FIELDS), jnp.int32),
        pltpu.VMEM((_RPW, FIELDS, DIM), jnp.float32),
    ],
    compiler_params=pltpu.CompilerParams(
        use_tc_tiling_on_sc=False, needs_layout_passes=False),
)
def _gather_kernel(table_hbm, idx_hbm, out_hbm, tab_v, idx_v, out_v):
    wid = lax.axis_index("s") * _NC + lax.axis_index("c")
    base = wid * _RPW
    pltpu.sync_copy(table_hbm, tab_v)
    pltpu.sync_copy(idx_hbm.at[pl.ds(base, _RPW)], idx_v)

    lanes = lax.iota(jnp.int32, DIM)
    zero = jnp.zeros((DIM,), jnp.int32)
    cols = [jnp.full((DIM,), c, jnp.int32) for c in range(DIM)]

    @plsc.parallel_loop(0, _GROUPS, unroll=4)
    def group(g):
        t = g * DIM + lanes
        v = plsc.load_gather(idx_v, [zero, t])
        for c in range(DIM):
            w = plsc.load_gather(tab_v, [v, cols[c]])
            plsc.store_scatter(out_v, [zero, t, cols[c]], w)
    pltpu.sync_copy(out_v, out_hbm.at[pl.ds(base, _RPW)])


def kernel(table, token_ids):
    return _gather_kernel(table, token_ids)


# speedup vs baseline: 11.5063x; 2.4799x over previous
import functools

import jax
import jax.numpy as jnp
from jax import lax
from jax.experimental import pallas as pl
from jax.experimental.pallas import tpu as pltpu
from jax.experimental.pallas import tpu_sc as plsc

VOCAB = 50
DIM = 16
BATCH = 4096
FIELDS = 26

_NC = 2
_NS = 16
_NW = _NC * _NS

_RPW = BATCH // _NW
_GPR = _RPW // DIM
_NITER = FIELDS * _GPR
_CT = DIM // 8
_ROWS = FIELDS * _CT * _NW * 8


@functools.partial(
    pl.kernel,
    out_type=jax.ShapeDtypeStruct((_ROWS, 128), jnp.float32),
    mesh=plsc.VectorSubcoreMesh(core_axis_name="c", subcore_axis_name="s"),
    scratch_types=[
        pltpu.VMEM((VOCAB, DIM), jnp.float32),
        pltpu.VMEM((_RPW, FIELDS), jnp.int32),
        pltpu.VMEM((FIELDS * DIM, _RPW), jnp.float32),
    ],
    compiler_params=pltpu.CompilerParams(
        use_tc_tiling_on_sc=False, needs_layout_passes=False),
)
def _gather_kernel(table_hbm, idx_hbm, out_hbm, tab_v, idx_v, out_v):
    wid = lax.axis_index("s") * _NC + lax.axis_index("c")
    pltpu.sync_copy(table_hbm, tab_v)
    pltpu.sync_copy(idx_hbm.at[pl.ds(wid * _RPW, _RPW)], idx_v)

    lanes26 = lax.iota(jnp.int32, DIM) * FIELDS
    zero = jnp.zeros((DIM,), jnp.int32)
    cols = [jnp.full((DIM,), c, jnp.int32) for c in range(DIM)]

    @plsc.parallel_loop(0, _NITER, unroll=4)
    def step(i):
        f = lax.shift_right_logical(i, 3)
        gr = lax.bitwise_and(i, 7)
        rb = gr * (DIM * FIELDS) + f
        v = plsc.load_gather(idx_v, [zero, lanes26 + rb])
        for c in range(DIM):
            w = plsc.load_gather(tab_v, [v, cols[c]])
            out_v[f * DIM + c, pl.ds(gr * DIM, DIM)] = w

    for k in range(FIELDS * _CT):
        pltpu.sync_copy(out_v.at[pl.ds(k * 8, 8)],
                        out_hbm.at[pl.ds(k * (_NW * 8) + wid * 8, 8)])


def kernel(table, token_ids):
    raw = _gather_kernel(table, token_ids)
    out = raw.reshape(FIELDS, _CT, _NW, 8, _RPW)
    return out.transpose(2, 4, 0, 1, 3).reshape(BATCH, FIELDS, DIM)


# speedup vs baseline: 12.5343x; 1.0893x over previous
import functools

import jax
import jax.numpy as jnp
from jax import lax
from jax.experimental import pallas as pl
from jax.experimental.pallas import tpu as pltpu
from jax.experimental.pallas import tpu_sc as plsc

VOCAB = 50
DIM = 16
BATCH = 4096
FIELDS = 26

_NC = 2
_NS = 16
_NW = _NC * _NS

_RPW = BATCH // _NW
_GPR = _RPW // DIM
_NITER = FIELDS * _GPR
_CT = DIM // 8
_ROWS = FIELDS * _CT * _NW * 8


@functools.partial(
    pl.kernel,
    out_type=jax.ShapeDtypeStruct((_ROWS, 128), jnp.float32),
    mesh=plsc.VectorSubcoreMesh(core_axis_name="c", subcore_axis_name="s"),
    scratch_types=[
        pltpu.VMEM((VOCAB, DIM), jnp.float32),
        pltpu.VMEM((_RPW, FIELDS), jnp.int32),
        pltpu.VMEM((FIELDS * DIM, _RPW), jnp.float32),
        pltpu.SemaphoreType.DMA,
        pltpu.SemaphoreType.DMA,
    ],
    compiler_params=pltpu.CompilerParams(
        use_tc_tiling_on_sc=False, needs_layout_passes=False),
)
def _gather_kernel(table_hbm, idx_hbm, out_hbm, tab_v, idx_v, out_v,
                   in_sem, out_sem):
    wid = lax.axis_index("s") * _NC + lax.axis_index("c")
    c_tab = pltpu.async_copy(table_hbm, tab_v, in_sem)
    c_idx = pltpu.async_copy(idx_hbm.at[pl.ds(wid * _RPW, _RPW)], idx_v,
                             in_sem)
    c_tab.wait()
    c_idx.wait()

    lanes26 = lax.iota(jnp.int32, DIM) * FIELDS
    zero = jnp.zeros((DIM,), jnp.int32)
    cols = [jnp.full((DIM,), c, jnp.int32) for c in range(DIM)]

    @plsc.parallel_loop(0, _NITER, unroll=4)
    def step(i):
        f = lax.shift_right_logical(i, 3)
        gr = lax.bitwise_and(i, 7)
        rb = gr * (DIM * FIELDS) + f
        v = plsc.load_gather(idx_v, [zero, lanes26 + rb])
        for c in range(DIM):
            w = plsc.load_gather(tab_v, [v, cols[c]])
            out_v[f * DIM + c, pl.ds(gr * DIM, DIM)] = w

    copies = [
        pltpu.async_copy(out_v.at[pl.ds(k * 8, 8)],
                         out_hbm.at[pl.ds(k * (_NW * 8) + wid * 8, 8)],
                         out_sem)
        for k in range(FIELDS * _CT)
    ]
    for c in copies:
        c.wait()


def kernel(table, token_ids):
    raw = _gather_kernel(table, token_ids)
    out = raw.reshape(FIELDS, _CT, _NW, 8, _RPW)
    return out.transpose(2, 4, 0, 1, 3).reshape(BATCH, FIELDS, DIM)


# speedup vs baseline: 12.5678x; 1.0027x over previous
import functools

import jax
import jax.numpy as jnp
from jax import lax
from jax.experimental import pallas as pl
from jax.experimental.pallas import tpu as pltpu
from jax.experimental.pallas import tpu_sc as plsc

VOCAB = 50
DIM = 16
BATCH = 4096
FIELDS = 26

_NC = 2
_NS = 16
_NW = _NC * _NS

_RPW = BATCH // _NW
_GPR = _RPW // DIM
_NITER = FIELDS * _GPR
_CT = DIM // 8
_ROWS = FIELDS * _CT * _NW * 8


@functools.partial(
    pl.kernel,
    out_type=jax.ShapeDtypeStruct((_ROWS, 128), jnp.float32),
    mesh=plsc.VectorSubcoreMesh(core_axis_name="c", subcore_axis_name="s"),
    scratch_types=[
        pltpu.VMEM((VOCAB, DIM), jnp.float32),
        pltpu.VMEM((_RPW, FIELDS), jnp.int32),
        pltpu.VMEM((FIELDS * DIM, _RPW), jnp.float32),
        pltpu.SemaphoreType.DMA,
        pltpu.SemaphoreType.DMA,
    ],
    compiler_params=pltpu.CompilerParams(
        use_tc_tiling_on_sc=False, needs_layout_passes=False),
)
def _gather_kernel(table_hbm, idx_hbm, out_hbm, tab_v, idx_v, out_v,
                   in_sem, out_sem):
    wid = lax.axis_index("s") * _NC + lax.axis_index("c")
    with jax.named_scope("stage_in"):
        c_tab = pltpu.async_copy(table_hbm, tab_v, in_sem)
        c_idx = pltpu.async_copy(idx_hbm.at[pl.ds(wid * _RPW, _RPW)], idx_v,
                                 in_sem)
        c_tab.wait()
        c_idx.wait()

    lanes26 = lax.iota(jnp.int32, DIM) * FIELDS
    zero = jnp.zeros((DIM,), jnp.int32)
    cols = [jnp.full((DIM,), c, jnp.int32) for c in range(DIM)]

    with jax.named_scope("gather"):
        @plsc.parallel_loop(0, _NITER, unroll=4)
        def step(i):
            f = lax.shift_right_logical(i, 3)
            gr = lax.bitwise_and(i, 7)
            rb = gr * (DIM * FIELDS) + f
            v = plsc.load_gather(idx_v, [zero, lanes26 + rb])
            for c in range(DIM):
                w = plsc.load_gather(tab_v, [v, cols[c]])
                out_v[f * DIM + c, pl.ds(gr * DIM, DIM)] = w

    with jax.named_scope("stream_out"):
        copies = [
            pltpu.async_copy(out_v.at[pl.ds(k * 8, 8)],
                             out_hbm.at[pl.ds(k * (_NW * 8) + wid * 8, 8)],
                             out_sem)
            for k in range(FIELDS * _CT)
        ]
        for c in copies:
            c.wait()


def kernel(table, token_ids):
    raw = _gather_kernel(table, token_ids)
    out = raw.reshape(FIELDS, _CT, _NW, 8, _RPW)
    return out.transpose(2, 4, 0, 1, 3).reshape(BATCH, FIELDS, DIM)
